# baseline (speedup 1.0000x reference)
import functools

import jax
import jax.numpy as jnp
from jax import lax
from jax.experimental import pallas as pl
from jax.experimental.pallas import tpu as pltpu
from jax.experimental.pallas import tpu_sc as plsc

B, C = 4096, 200
ENC_DIM, TOK_DIM, HIDDEN = 128, 64, 512
NW = 32
RPW = B // NW
G1, G2 = 104, 96
UNROLL = 8


def _q_body(enc_ref, w1_ref, b1_ref, w2_ref, q_ref):
    h = lax.dot_general(enc_ref[...], w1_ref[...], (((1,), (1,)), ((), ())),
                        preferred_element_type=jnp.float32)
    h = h + b1_ref[...]
    h = 0.5 * h * (1.0 + lax.erf(h * 0.7071067811865476))
    q_ref[...] = lax.dot_general(h, w2_ref[...], (((1,), (1,)), ((), ())),
                                 preferred_element_type=jnp.float32)


def _q_proj(enc_vec, W1, b1, W2):
    return pl.pallas_call(
        _q_body,
        grid=(8,),
        in_specs=[
            pl.BlockSpec((512, ENC_DIM), lambda i: (i, 0)),
            pl.BlockSpec((HIDDEN, ENC_DIM), lambda i: (0, 0)),
            pl.BlockSpec((1, HIDDEN), lambda i: (0, 0)),
            pl.BlockSpec((TOK_DIM, HIDDEN), lambda i: (0, 0)),
        ],
        out_specs=pl.BlockSpec((512, TOK_DIM), lambda i: (i, 0)),
        out_shape=jax.ShapeDtypeStruct((B, TOK_DIM), jnp.float32),
    )(enc_vec, W1, b1.reshape(1, HIDDEN), W2)


def _score_body(q_hbm, cand_hbm, emb_hbm, out_hbm, idx_v, q_v, buf, out_v, sem):
    wid = lax.axis_index("s") * 2 + lax.axis_index("c")
    base = wid * RPW
    pltpu.sync_copy(cand_hbm.at[pl.ds(base, RPW)], idx_v)
    pltpu.sync_copy(q_hbm.at[pl.ds(base, RPW)], q_v)
    lane = lax.iota(jnp.int32, 16)
    mask15 = lane == 15

    def row_body(b, carry):
        cp1 = pltpu.async_copy(
            emb_hbm.at[idx_v.at[b, pl.ds(0, G1)]], buf.at[pl.ds(0, G1)], sem)
        cp2 = pltpu.async_copy(
            emb_hbm.at[idx_v.at[b, pl.ds(G1, G2)]], buf.at[pl.ds(G1, G2)], sem)
        cp1.wait()
        cp2.wait()
        qv = tuple(q_v[b, pl.ds(16 * j, 16)] for j in range(TOK_DIM // 16))
        out_base = b * C

        def chunk_body(t, carry2):
            c0 = t * UNROLL
            for u in range(UNROLL):
                c = c0 + u
                prod = buf[c, pl.ds(0, 16)] * qv[0]
                for j in range(1, TOK_DIM // 16):
                    prod = prod + buf[c, pl.ds(16 * j, 16)] * qv[j]
                s = plsc.cumsum(prod)
                plsc.store_scatter(
                    out_v, [jnp.full((16,), out_base + c, jnp.int32)], s,
                    mask=mask15)
            return carry2

        lax.fori_loop(0, C // UNROLL, chunk_body, 0)
        return carry

    lax.fori_loop(0, RPW, row_body, 0)
    pltpu.sync_copy(out_v, out_hbm.at[pl.ds(base * C, RPW * C)])


@functools.lru_cache(maxsize=1)
def _make_score_kernel():
    mesh = plsc.VectorSubcoreMesh(core_axis_name="c", subcore_axis_name="s")
    return pl.kernel(
        _score_body,
        out_type=jax.ShapeDtypeStruct((B * C,), jnp.float32),
        mesh=mesh,
        scratch_types=[
            pltpu.VMEM((RPW, C), jnp.int32),
            pltpu.VMEM((RPW, TOK_DIM), jnp.float32),
            pltpu.VMEM((C, TOK_DIM), jnp.float32),
            pltpu.VMEM((RPW * C,), jnp.float32),
            pltpu.SemaphoreType.DMA,
        ],
        compiler_params=pltpu.CompilerParams(
            use_tc_tiling_on_sc=False, needs_layout_passes=False),
    )


def kernel(enc_vec, cand_tok, tok_emb, W1, b1, W2):
    q = _q_proj(enc_vec, W1, b1, W2)
    cand = cand_tok.astype(jnp.int32)
    flat = _make_score_kernel()(q, cand, tok_emb)
    return flat.reshape(B, C)


# speedup vs baseline: 1.1400x; 1.1400x over previous
import functools

import jax
import jax.numpy as jnp
from jax import lax
from jax.experimental import pallas as pl
from jax.experimental.pallas import tpu as pltpu
from jax.experimental.pallas import tpu_sc as plsc

B, C = 4096, 200
ENC_DIM, TOK_DIM, HIDDEN = 128, 64, 512
NW = 32
RPW = B // NW
G1, G2 = 104, 96
UNROLL = 8


def _q_body(enc_ref, w1_ref, b1_ref, w2_ref, q_ref):
    h = lax.dot_general(enc_ref[...], w1_ref[...], (((1,), (1,)), ((), ())),
                        preferred_element_type=jnp.float32)
    h = h + b1_ref[...]
    h = 0.5 * h * (1.0 + lax.erf(h * 0.7071067811865476))
    q_ref[...] = lax.dot_general(h, w2_ref[...], (((1,), (1,)), ((), ())),
                                 preferred_element_type=jnp.float32)


def _q_proj(enc_vec, W1, b1, W2):
    return pl.pallas_call(
        _q_body,
        grid=(8,),
        in_specs=[
            pl.BlockSpec((512, ENC_DIM), lambda i: (i, 0)),
            pl.BlockSpec((HIDDEN, ENC_DIM), lambda i: (0, 0)),
            pl.BlockSpec((1, HIDDEN), lambda i: (0, 0)),
            pl.BlockSpec((TOK_DIM, HIDDEN), lambda i: (0, 0)),
        ],
        out_specs=pl.BlockSpec((512, TOK_DIM), lambda i: (i, 0)),
        out_shape=jax.ShapeDtypeStruct((B, TOK_DIM), jnp.float32),
    )(enc_vec, W1, b1.reshape(1, HIDDEN), W2)


def _score_body(q_hbm, cand_hbm, emb_hbm, out_hbm, idx_v, q_v, buf_a, buf_b,
                out_v, sem_a, sem_b):
    wid = lax.axis_index("s") * 2 + lax.axis_index("c")
    base = wid * RPW
    pltpu.sync_copy(cand_hbm.at[pl.ds(base, RPW)], idx_v)
    pltpu.sync_copy(q_hbm.at[pl.ds(base, RPW)], q_v)
    lane = lax.iota(jnp.int32, 16)
    mask15 = lane == 15

    def gather(b, buf, sem):
        pltpu.async_copy(
            emb_hbm.at[idx_v.at[b, pl.ds(0, G1)]], buf.at[pl.ds(0, G1)], sem)
        pltpu.async_copy(
            emb_hbm.at[idx_v.at[b, pl.ds(G1, G2)]], buf.at[pl.ds(G1, G2)], sem)

    def drain(buf, sem):
        pltpu.make_async_copy(emb_hbm.at[pl.ds(0, C)], buf, sem).wait()

    def compute(b, buf):
        qv = tuple(q_v[b, pl.ds(16 * j, 16)] for j in range(TOK_DIM // 16))
        out_base = b * C

        def chunk_body(t, carry2):
            c0 = t * UNROLL
            for u in range(UNROLL):
                c = c0 + u
                prod = buf[c, pl.ds(0, 16)] * qv[0]
                for j in range(1, TOK_DIM // 16):
                    prod = prod + buf[c, pl.ds(16 * j, 16)] * qv[j]
                s = plsc.cumsum(prod)
                plsc.store_scatter(
                    out_v, [jnp.full((16,), out_base + c, jnp.int32)], s,
                    mask=mask15)
            return carry2

        lax.fori_loop(0, C // UNROLL, chunk_body, 0)

    def pair_body(t, carry):
        b0 = 2 * t
        gather(b0 + 1, buf_b, sem_b)
        drain(buf_a, sem_a)
        compute(b0, buf_a)
        gather(jnp.minimum(b0 + 2, RPW - 1), buf_a, sem_a)
        drain(buf_b, sem_b)
        compute(b0 + 1, buf_b)
        return carry

    gather(0, buf_a, sem_a)
    lax.fori_loop(0, RPW // 2, pair_body, 0)
    drain(buf_a, sem_a)
    pltpu.sync_copy(out_v, out_hbm.at[pl.ds(base * C, RPW * C)])


@functools.lru_cache(maxsize=1)
def _make_score_kernel():
    mesh = plsc.VectorSubcoreMesh(core_axis_name="c", subcore_axis_name="s")
    return pl.kernel(
        _score_body,
        out_type=jax.ShapeDtypeStruct((B * C,), jnp.float32),
        mesh=mesh,
        scratch_types=[
            pltpu.VMEM((RPW, C), jnp.int32),
            pltpu.VMEM((RPW, TOK_DIM), jnp.float32),
            pltpu.VMEM((C, TOK_DIM), jnp.float32),
            pltpu.VMEM((C, TOK_DIM), jnp.float32),
            pltpu.VMEM((RPW * C,), jnp.float32),
            pltpu.SemaphoreType.DMA,
            pltpu.SemaphoreType.DMA,
        ],
        compiler_params=pltpu.CompilerParams(
            use_tc_tiling_on_sc=False, needs_layout_passes=False),
    )


def kernel(enc_vec, cand_tok, tok_emb, W1, b1, W2):
    q = _q_proj(enc_vec, W1, b1, W2)
    cand = cand_tok.astype(jnp.int32)
    flat = _make_score_kernel()(q, cand, tok_emb)
    return flat.reshape(B, C)
